# baseline (speedup 1.0000x reference)
import functools

import jax
import jax.numpy as jnp
from jax import lax
from jax.experimental import pallas as pl
from jax.experimental.pallas import tpu as pltpu
from jax.experimental.pallas import tpu_sc as plsc

B = 16384
D = 64
NC = 2
NS = 16
NW = NC * NS
BPW = B // NW
CHUNK = 128
NCH = BPW // CHUNK

_MESH = plsc.VectorSubcoreMesh(core_axis_name="c", subcore_axis_name="s")


def _sc_gather_body(uid_hbm, iid_hbm, gu_t, gi_t, mu_t, mi_t,
                    out_gu, out_gi, out_mu, out_mi,
                    idx_u, idx_i, rows0, rows1, sem0, sem1):
    wid = lax.axis_index("s") * NC + lax.axis_index("c")
    base = wid * BPW

    pltpu.sync_copy(uid_hbm.at[wid], idx_u)
    pltpu.sync_copy(iid_hbm.at[wid], idx_i)

    tables = (gu_t, gi_t, mu_t, mi_t)
    idxs = (idx_u, idx_i, idx_u, idx_i)
    outs = (out_gu, out_gi, out_mu, out_mi)
    bufs = (rows0, rows1, rows0, rows1)
    sems = (sem0, sem1, sem0, sem1)

    def fire(t):
        for j in range(NCH):
            pltpu.async_copy(
                tables[t].at[idxs[t].at[j]],
                bufs[t].at[pl.ds(j * CHUNK, CHUNK)],
                sems[t],
            )

    def drain(t):
        for j in range(NCH):
            pltpu.make_async_copy(
                tables[t].at[idxs[t].at[j]],
                bufs[t].at[pl.ds(j * CHUNK, CHUNK)],
                sems[t],
            ).wait()

    fire(0)
    fire(1)
    for t in range(4):
        drain(t)
        if t + 2 < 4:
            pltpu.sync_copy(bufs[t], outs[t].at[pl.ds(base, BPW)])
            fire(t + 2)
        else:
            pltpu.sync_copy(bufs[t], outs[t].at[pl.ds(base, BPW)])


@functools.partial(
    pl.kernel,
    out_type=[jax.ShapeDtypeStruct((B, D), jnp.float32)] * 4,
    mesh=_MESH,
    scratch_types=[
        pltpu.VMEM((NCH, CHUNK), jnp.int32),
        pltpu.VMEM((NCH, CHUNK), jnp.int32),
        pltpu.VMEM((BPW, D), jnp.float32),
        pltpu.VMEM((BPW, D), jnp.float32),
        pltpu.SemaphoreType.DMA,
        pltpu.SemaphoreType.DMA,
    ],
    compiler_params=pltpu.CompilerParams(use_tc_tiling_on_sc=False),
)
def _sc_gather(*args):
    _sc_gather_body(*args)


ROWS = 512
GRID = B // ROWS


def _tc_body(gu, gv, mu, mv, w0u, w0v, b0, w1, b1, w2, b2, wg, wh, bo, out):
    f32 = jnp.float32
    h = jnp.dot(mu[...], w0u[...], preferred_element_type=f32)
    h += jnp.dot(mv[...], w0v[...], preferred_element_type=f32)
    h = jnp.maximum(h + b0[...], 0.0)
    h = jnp.maximum(jnp.dot(h, w1[...], preferred_element_type=f32) + b1[...], 0.0)
    h = jnp.maximum(jnp.dot(h, w2[...], preferred_element_type=f32) + b2[...], 0.0)
    gmf = gu[...] * gv[...]
    o = jnp.dot(gmf, wg[...], preferred_element_type=f32)
    o += jnp.dot(h, wh[...], preferred_element_type=f32)
    out[...] = o + bo[...]


def _tc_dense(gu, gv, mu, mv, w0u, w0v, b0, w1, b1, w2, b2, wg, wh, bo):
    row_spec = pl.BlockSpec((ROWS, D), lambda i: (i, 0))
    full = lambda shape: pl.BlockSpec(shape, lambda i: tuple(0 for _ in shape))
    return pl.pallas_call(
        _tc_body,
        grid=(GRID,),
        in_specs=[
            row_spec, row_spec, row_spec, row_spec,
            full((D, 256)), full((D, 256)), full((1, 256)),
            full((256, 128)), full((1, 128)),
            full((128, D)), full((1, D)),
            full((D, 1)), full((D, 1)), full((1, 1)),
        ],
        out_specs=pl.BlockSpec((ROWS, 1), lambda i: (i, 0)),
        out_shape=jax.ShapeDtypeStruct((B, 1), jnp.float32),
    )(gu, gv, mu, mv, w0u, w0v, b0, w1, b1, w2, b2, wg, wh, bo)


def kernel(user_ids, item_ids, gmf_user_emb, gmf_item_emb, mlp_user_emb,
           mlp_item_emb, W0, b0, W1, b1, W2, b2, W_out, b_out):
    uid = user_ids.astype(jnp.int32).reshape(NW, NCH, CHUNK)
    iid = item_ids.astype(jnp.int32).reshape(NW, NCH, CHUNK)
    gu, gv, mu, mv = _sc_gather(uid, iid, gmf_user_emb, gmf_item_emb,
                                mlp_user_emb, mlp_item_emb)
    w0u = W0[:D]
    w0v = W0[D:]
    wg = W_out[:D]
    wh = W_out[D:]
    out = _tc_dense(gu, gv, mu, mv, w0u, w0v, b0.reshape(1, -1),
                    W1, b1.reshape(1, -1), W2, b2.reshape(1, -1),
                    wg, wh, b_out.reshape(1, 1))
    return out.reshape(B)


# speedup vs baseline: 1.0026x; 1.0026x over previous
import functools

import jax
import jax.numpy as jnp
from jax import lax
from jax.experimental import pallas as pl
from jax.experimental.pallas import tpu as pltpu
from jax.experimental.pallas import tpu_sc as plsc

B = 16384
D = 64
VP = 500000
NC = 2
NS = 16
NW = NC * NS
BPW = B // NW
CHUNK = 128
HALF = 256

_MESH = plsc.VectorSubcoreMesh(core_axis_name="c", subcore_axis_name="s")


def _sc_gather_body(pidx_u_hbm, pidx_i_hbm, gu_t, gi_t, mu_t, mi_t,
                    out_gu, out_gi, out_mu, out_mi,
                    idx_u, idx_i, rows0, rows1, sem0, sem1):
    wid = lax.axis_index("s") * NC + lax.axis_index("c")
    base = wid * BPW

    pltpu.sync_copy(pidx_u_hbm.at[pl.ds(base, BPW)], idx_u)
    pltpu.sync_copy(pidx_i_hbm.at[pl.ds(base, BPW)], idx_i)

    units = []
    for t, (tab, idx, out) in enumerate((
            (gu_t, idx_u, out_gu), (gi_t, idx_i, out_gi),
            (mu_t, idx_u, out_mu), (mi_t, idx_i, out_mi))):
        for h in range(2):
            units.append((tab, idx, out, h))
    bufs = (rows0, rows1)
    sems = (sem0, sem1)

    def fire(u):
        tab, idx, _, h = units[u]
        for j in range(2):
            pltpu.async_copy(
                tab.at[idx.at[pl.ds(h * HALF + j * CHUNK, CHUNK)]],
                bufs[u % 2].at[pl.ds(j * CHUNK, CHUNK)],
                sems[u % 2],
            )

    def drain(u):
        tab, idx, _, h = units[u]
        for j in range(2):
            pltpu.make_async_copy(
                tab.at[idx.at[pl.ds(h * HALF + j * CHUNK, CHUNK)]],
                bufs[u % 2].at[pl.ds(j * CHUNK, CHUNK)],
                sems[u % 2],
            ).wait()

    fire(0)
    fire(1)
    for u in range(8):
        drain(u)
        _, _, out, h = units[u]
        pltpu.sync_copy(bufs[u % 2], out.at[pl.ds(base + h * HALF, HALF)])
        if u + 2 < 8:
            fire(u + 2)


@functools.partial(
    pl.kernel,
    out_type=[jax.ShapeDtypeStruct((B, 2 * D), jnp.float32)] * 4,
    mesh=_MESH,
    scratch_types=[
        pltpu.VMEM((BPW,), jnp.int32),
        pltpu.VMEM((BPW,), jnp.int32),
        pltpu.VMEM((HALF, 2 * D), jnp.float32),
        pltpu.VMEM((HALF, 2 * D), jnp.float32),
        pltpu.SemaphoreType.DMA,
        pltpu.SemaphoreType.DMA,
    ],
)
def _sc_gather(*args):
    _sc_gather_body(*args)


ROWS = 512
GRID = B // ROWS


def _tc_body(pu, pv, gu, gv, mu, mv, w0u, w0v, b0, w1, b1, w2, b2, wg, wh,
             bo, out):
    f32 = jnp.float32
    su = pu[...] != 0
    sv = pv[...] != 0

    def half(x, sel):
        return jnp.where(sel, x[:, D:], x[:, :D])

    hu = half(mu[...], su)
    hv = half(mv[...], sv)
    h = jnp.dot(hu, w0u[...], preferred_element_type=f32)
    h += jnp.dot(hv, w0v[...], preferred_element_type=f32)
    h = jnp.maximum(h + b0[...], 0.0)
    h = jnp.maximum(jnp.dot(h, w1[...], preferred_element_type=f32) + b1[...], 0.0)
    h = jnp.maximum(jnp.dot(h, w2[...], preferred_element_type=f32) + b2[...], 0.0)
    gmf = half(gu[...], su) * half(gv[...], sv)
    o = jnp.dot(gmf, wg[...], preferred_element_type=f32)
    o += jnp.dot(h, wh[...], preferred_element_type=f32)
    out[...] = o + bo[...]


def _tc_dense(pu, pv, gu, gv, mu, mv, w0u, w0v, b0, w1, b1, w2, b2, wg, wh, bo):
    row_spec = pl.BlockSpec((ROWS, 2 * D), lambda i: (i, 0))
    par_spec = pl.BlockSpec((ROWS, 1), lambda i: (i, 0))
    full = lambda shape: pl.BlockSpec(shape, lambda i: tuple(0 for _ in shape))
    return pl.pallas_call(
        _tc_body,
        grid=(GRID,),
        in_specs=[
            par_spec, par_spec,
            row_spec, row_spec, row_spec, row_spec,
            full((D, 256)), full((D, 256)), full((1, 256)),
            full((256, 128)), full((1, 128)),
            full((128, D)), full((1, D)),
            full((D, 1)), full((D, 1)), full((1, 1)),
        ],
        out_specs=pl.BlockSpec((ROWS, 1), lambda i: (i, 0)),
        out_shape=jax.ShapeDtypeStruct((B, 1), jnp.float32),
    )(pu, pv, gu, gv, mu, mv, w0u, w0v, b0, w1, b1, w2, b2, wg, wh, bo)


def kernel(user_ids, item_ids, gmf_user_emb, gmf_item_emb, mlp_user_emb,
           mlp_item_emb, W0, b0, W1, b1, W2, b2, W_out, b_out):
    uid = user_ids.astype(jnp.int32)
    iid = item_ids.astype(jnp.int32)
    pidx_u = lax.shift_right_logical(uid, 1)
    pidx_i = lax.shift_right_logical(iid, 1)
    par_u = (uid & 1).reshape(B, 1)
    par_i = (iid & 1).reshape(B, 1)
    packed = [t.reshape(VP, 2 * D) for t in
              (gmf_user_emb, gmf_item_emb, mlp_user_emb, mlp_item_emb)]
    gu, gv, mu, mv = _sc_gather(pidx_u, pidx_i, *packed)
    w0u = W0[:D]
    w0v = W0[D:]
    wg = W_out[:D]
    wh = W_out[D:]
    out = _tc_dense(par_u, par_i, gu, gv, mu, mv, w0u, w0v, b0.reshape(1, -1),
                    W1, b1.reshape(1, -1), W2, b2.reshape(1, -1),
                    wg, wh, b_out.reshape(1, 1))
    return out.reshape(B)


# speedup vs baseline: 3.7287x; 3.7191x over previous
import functools

import jax
import jax.numpy as jnp
from jax import lax
from jax.experimental import pallas as pl
from jax.experimental.pallas import tpu as pltpu
from jax.experimental.pallas import tpu_sc as plsc

B = 16384
D = 64
V = 1000000
NC = 2
NS = 16
NW = NC * NS
POS = B // NW
GROUP = 256
CW = 256
VMAIN = (V // 128) * 128 - 64
CAP_BLK = (V - CW) // 128
TAIL0 = CAP_BLK * 128 + CW
TAILW = V - TAIL0

INT_MIN = -2147483648
SENT = 2 ** 30

_MESH = plsc.VectorSubcoreMesh(core_axis_name="c", subcore_axis_name="s")


def _extract(ref, p):
    start = pl.multiple_of((p // 16) * 16, 16)
    vec = ref[pl.ds(start, 16)]
    m = lax.broadcasted_iota(jnp.int32, (16,), 0) == (p - start)
    return jnp.max(jnp.where(m, vec, jnp.int32(INT_MIN)))


def _scan_body(sidx_hbm, perm_hbm, ta, tb, out_hbm,
               idx_v, pidx_v, ba0, ba1, bb0, bb1, outbuf, tla, tlb,
               sa0, sa1, sb0, sb1):
    wid = lax.axis_index("s") * NC + lax.axis_index("c")
    base = wid * POS

    pltpu.sync_copy(sidx_hbm.at[pl.ds(base, POS)], idx_v.at[pl.ds(0, POS)])
    idx_v[pl.ds(POS, 16)] = jnp.full((16,), SENT, jnp.int32)
    pltpu.sync_copy(perm_hbm.at[pl.ds(4 * wid, 4)], pidx_v)
    pltpu.sync_copy(ta.at[:, pl.ds(TAIL0, TAILW)], tla)
    pltpu.sync_copy(tb.at[:, pl.ds(TAIL0, TAILW)], tlb)

    dvec = lax.broadcasted_iota(jnp.int32, (16,), 0)

    def chunk_col(c_lo, c):
        blk = jnp.minimum(c_lo + 2 * c, CAP_BLK)
        return pl.multiple_of(blk * 128, 128)

    def fire(col, buf_a, buf_b, sem_a, sem_b):
        pltpu.async_copy(ta.at[:, pl.ds(col, CW)], buf_a, sem_a)
        pltpu.async_copy(tb.at[:, pl.ds(col, CW)], buf_b, sem_b)

    def drain(col, buf_a, buf_b, sem_a, sem_b):
        pltpu.make_async_copy(ta.at[:, pl.ds(col, CW)], buf_a, sem_a).wait()
        pltpu.make_async_copy(tb.at[:, pl.ds(col, CW)], buf_b, sem_b).wait()

    def emit(ix, p, gbase, col0, buf_a, buf_b):
        row = p - (base + gbase)
        colv = jnp.full((16,), ix - col0, jnp.int32)
        for k in range(4):
            va = plsc.load_gather(buf_a, (dvec + 16 * k, colv))
            outbuf[row, pl.ds(16 * k, 16)] = va
            vb = plsc.load_gather(buf_b, (dvec + 16 * k, colv))
            outbuf[row, pl.ds(D + 16 * k, 16)] = vb

    def consume(carry, gbase, hi_col, col0, buf_a, buf_b):
        gend = base + gbase + GROUP

        def cond(c):
            p, ix = c
            return jnp.logical_and(p < gend, ix < hi_col)

        def body(c):
            p, ix = c
            emit(ix, p, gbase, col0, buf_a, buf_b)
            return (p + 1, _extract(idx_v, p + 1 - base))

        return lax.while_loop(cond, body, carry)

    for g in range(2):
        gbase = g * GROUP
        first = _extract(idx_v, gbase)
        last = _extract(idx_v, gbase + GROUP - 1)
        c_lo = jnp.minimum(first >> 7, CAP_BLK)
        c_hi = jnp.minimum(last >> 7, CAP_BLK)
        nchk = (c_hi - c_lo) // 2 + 1
        npair = (nchk + 1) // 2

        fire(chunk_col(c_lo, 0), ba0, bb0, sa0, sb0)
        fire(chunk_col(c_lo, 1), ba1, bb1, sa1, sb1)

        p0 = base + gbase
        ix0 = _extract(idx_v, gbase)

        def pair_body(j, carry, c_lo=c_lo, gbase=gbase):
            col_e = chunk_col(c_lo, 2 * j)
            col_o = chunk_col(c_lo, 2 * j + 1)
            drain(col_e, ba0, bb0, sa0, sb0)
            carry = consume(carry, gbase, col_e + CW, col_e, ba0, bb0)
            fire(chunk_col(c_lo, 2 * j + 2), ba0, bb0, sa0, sb0)
            drain(col_o, ba1, bb1, sa1, sb1)
            carry = consume(carry, gbase, col_o + CW, col_o, ba1, bb1)
            fire(chunk_col(c_lo, 2 * j + 3), ba1, bb1, sa1, sb1)
            return carry

        carry = lax.fori_loop(0, npair, pair_body, (p0, ix0))

        drain(chunk_col(c_lo, 2 * npair), ba0, bb0, sa0, sb0)
        drain(chunk_col(c_lo, 2 * npair + 1), ba1, bb1, sa1, sb1)

        gend = base + gbase + GROUP

        def tail_cond(c):
            p, _ = c
            return p < gend

        def tail_body(c, gbase=gbase):
            p, ix = c
            emit(ix, p, gbase, TAIL0, tla, tlb)
            return (p + 1, _extract(idx_v, p + 1 - base))

        lax.while_loop(tail_cond, tail_body, carry)

        pltpu.sync_copy(outbuf.at[pl.ds(0, 128)],
                        out_hbm.at[pidx_v.at[2 * g]])
        pltpu.sync_copy(outbuf.at[pl.ds(128, 128)],
                        out_hbm.at[pidx_v.at[2 * g + 1]])


@functools.partial(
    pl.kernel,
    out_type=jax.ShapeDtypeStruct((B, 2 * D), jnp.float32),
    mesh=_MESH,
    scratch_types=[
        pltpu.VMEM((POS + 16,), jnp.int32),
        pltpu.VMEM((4, 128), jnp.int32),
        pltpu.VMEM((D, CW), jnp.float32),
        pltpu.VMEM((D, CW), jnp.float32),
        pltpu.VMEM((D, CW), jnp.float32),
        pltpu.VMEM((D, CW), jnp.float32),
        pltpu.VMEM((GROUP, 2 * D), jnp.float32),
        pltpu.VMEM((D, TAILW), jnp.float32),
        pltpu.VMEM((D, TAILW), jnp.float32),
        pltpu.SemaphoreType.DMA,
        pltpu.SemaphoreType.DMA,
        pltpu.SemaphoreType.DMA,
        pltpu.SemaphoreType.DMA,
    ],
    compiler_params=pltpu.CompilerParams(needs_layout_passes=False),
)
def _sc_scan_gather(*args):
    _scan_body(*args)


ROWS = 512
GRID = B // ROWS


def _tc_body(eu, ev, w0u, w0v, b0, w1, b1, w2, b2, wg, wh, bo, out):
    f32 = jnp.float32
    eu_ = eu[...]
    ev_ = ev[...]
    h = jnp.dot(eu_[:, D:], w0u[...], preferred_element_type=f32)
    h += jnp.dot(ev_[:, D:], w0v[...], preferred_element_type=f32)
    h = jnp.maximum(h + b0[...], 0.0)
    h = jnp.maximum(jnp.dot(h, w1[...], preferred_element_type=f32) + b1[...], 0.0)
    h = jnp.maximum(jnp.dot(h, w2[...], preferred_element_type=f32) + b2[...], 0.0)
    gmf = eu_[:, :D] * ev_[:, :D]
    o = jnp.dot(gmf, wg[...], preferred_element_type=f32)
    o += jnp.dot(h, wh[...], preferred_element_type=f32)
    out[...] = o + bo[...]


def _tc_dense(eu, ev, w0u, w0v, b0, w1, b1, w2, b2, wg, wh, bo):
    row_spec = pl.BlockSpec((ROWS, 2 * D), lambda i: (i, 0))
    full = lambda shape: pl.BlockSpec(shape, lambda i: tuple(0 for _ in shape))
    return pl.pallas_call(
        _tc_body,
        grid=(GRID,),
        in_specs=[
            row_spec, row_spec,
            full((D, 256)), full((D, 256)), full((1, 256)),
            full((256, 128)), full((1, 128)),
            full((128, D)), full((1, D)),
            full((D, 1)), full((D, 1)), full((1, 1)),
        ],
        out_specs=pl.BlockSpec((ROWS, 1), lambda i: (i, 0)),
        out_shape=jax.ShapeDtypeStruct((B, 1), jnp.float32),
    )(eu, ev, w0u, w0v, b0, w1, b1, w2, b2, wg, wh, bo)


def kernel(user_ids, item_ids, gmf_user_emb, gmf_item_emb, mlp_user_emb,
           mlp_item_emb, W0, b0, W1, b1, W2, b2, W_out, b_out):
    uid = user_ids.astype(jnp.int32)
    iid = item_ids.astype(jnp.int32)
    order_u = jnp.argsort(uid).astype(jnp.int32)
    order_i = jnp.argsort(iid).astype(jnp.int32)
    su = jnp.take(uid, order_u)
    si = jnp.take(iid, order_i)
    perm_u = order_u.reshape(B // 128, 128)
    perm_i = order_i.reshape(B // 128, 128)
    eu = _sc_scan_gather(su, perm_u, gmf_user_emb.T, mlp_user_emb.T)
    ev = _sc_scan_gather(si, perm_i, gmf_item_emb.T, mlp_item_emb.T)
    w0u = W0[:D]
    w0v = W0[D:]
    wg = W_out[:D]
    wh = W_out[D:]
    out = _tc_dense(eu, ev, w0u, w0v, b0.reshape(1, -1),
                    W1, b1.reshape(1, -1), W2, b2.reshape(1, -1),
                    wg, wh, b_out.reshape(1, 1))
    return out.reshape(B)
